# Initial kernel scaffold; baseline (speedup 1.0000x reference)
import jax
import jax.numpy as jnp
from jax.experimental import pallas as pl


def kernel(x, W, b):
    raise NotImplementedError("write your pallas kernel here")


# speedup vs baseline: 5.0366x; 5.0366x over previous
import functools

import jax
import jax.numpy as jnp
from jax.experimental import pallas as pl
from jax.experimental.pallas import tpu as pltpu

_TOP_K = 8
_ROW_TILE = 512


def _gating_body(x_ref, w_ref, b_ref, o_ref):
    logits = jnp.dot(x_ref[...], w_ref[...],
                     preferred_element_type=jnp.float32) + b_ref[...]
    n_exp = logits.shape[-1]
    lane = jax.lax.broadcasted_iota(jnp.int32, logits.shape, 1)
    cur = logits
    mask = jnp.zeros(logits.shape, dtype=jnp.bool_)
    neg_inf = jnp.float32(-jnp.inf)
    for _ in range(_TOP_K):
        m = jnp.max(cur, axis=-1, keepdims=True)
        cand = jnp.where(cur == m, lane, n_exp)
        first = jnp.min(cand, axis=-1, keepdims=True)
        hit = lane == first
        mask = jnp.logical_or(mask, hit)
        cur = jnp.where(hit, neg_inf, cur)
    masked = jnp.where(mask, logits, 0.0)
    mx = jnp.max(masked, axis=-1, keepdims=True)
    e = jnp.exp(masked - mx)
    o_ref[...] = e / jnp.sum(e, axis=-1, keepdims=True)


@jax.jit
def kernel(x, W, b):
    n_tok, d_model = x.shape
    n_exp = W.shape[1]
    b2 = b.reshape(1, n_exp)
    grid = (n_tok // _ROW_TILE,)
    return pl.pallas_call(
        _gating_body,
        grid=grid,
        in_specs=[
            pl.BlockSpec((_ROW_TILE, d_model), lambda i: (i, 0)),
            pl.BlockSpec((d_model, n_exp), lambda i: (0, 0)),
            pl.BlockSpec((1, n_exp), lambda i: (0, 0)),
        ],
        out_specs=pl.BlockSpec((_ROW_TILE, n_exp), lambda i: (i, 0)),
        out_shape=jax.ShapeDtypeStruct((n_tok, n_exp), jnp.float32),
        compiler_params=pltpu.CompilerParams(
            dimension_semantics=("arbitrary",),
        ),
    )(x, W, b2)


# speedup vs baseline: 5.7149x; 1.1347x over previous
import functools

import jax
import jax.numpy as jnp
from jax.experimental import pallas as pl
from jax.experimental.pallas import tpu as pltpu

_TOP_K = 8
_ROW_TILE = 512


def _gating_body(x_ref, w_ref, b_ref, o_ref):
    logits = jnp.dot(x_ref[...], w_ref[...],
                     preferred_element_type=jnp.float32) + b_ref[...]
    n_exp = logits.shape[-1]
    lane = jax.lax.broadcasted_iota(
        jnp.int32, logits.shape, 1).astype(jnp.float32)
    big = jnp.float32(n_exp)
    cur = logits
    mask = jnp.zeros(logits.shape, dtype=jnp.bool_)
    neg_inf = jnp.float32(-jnp.inf)
    row_max = None
    for it in range(_TOP_K):
        m = jnp.max(cur, axis=-1, keepdims=True)
        if it == 0:
            row_max = m
        cand = jnp.where(cur == m, lane, big)
        first = jnp.min(cand, axis=-1, keepdims=True)
        hit = lane == first
        mask = jnp.logical_or(mask, hit)
        cur = jnp.where(hit, neg_inf, cur)
    masked = jnp.where(mask, logits, 0.0)
    mx = jnp.maximum(row_max, 0.0)
    e = jnp.exp(masked - mx)
    o_ref[...] = e / jnp.sum(e, axis=-1, keepdims=True)


@jax.jit
def kernel(x, W, b):
    n_tok, d_model = x.shape
    n_exp = W.shape[1]
    b2 = b.reshape(1, n_exp)
    grid = (n_tok // _ROW_TILE,)
    return pl.pallas_call(
        _gating_body,
        grid=grid,
        in_specs=[
            pl.BlockSpec((_ROW_TILE, d_model), lambda i: (i, 0)),
            pl.BlockSpec((d_model, n_exp), lambda i: (0, 0)),
            pl.BlockSpec((1, n_exp), lambda i: (0, 0)),
        ],
        out_specs=pl.BlockSpec((_ROW_TILE, n_exp), lambda i: (i, 0)),
        out_shape=jax.ShapeDtypeStruct((n_tok, n_exp), jnp.float32),
        compiler_params=pltpu.CompilerParams(
            dimension_semantics=("arbitrary",),
        ),
    )(x, W, b2)


# speedup vs baseline: 6.3790x; 1.1162x over previous
import functools

import jax
import jax.numpy as jnp
from jax.experimental import pallas as pl
from jax.experimental.pallas import tpu as pltpu

_TOP_K = 8
_ROW_TILE = 1024


def _gating_body(x_ref, w_ref, b_ref, o_ref):
    logits = jnp.dot(x_ref[...], w_ref[...],
                     preferred_element_type=jnp.float32) + b_ref[...]
    n_exp = logits.shape[-1]
    lane = jax.lax.broadcasted_iota(
        jnp.int32, logits.shape, 1).astype(jnp.float32)
    big = jnp.float32(n_exp)
    cur = logits
    mask = jnp.zeros(logits.shape, dtype=jnp.bool_)
    neg_inf = jnp.float32(-jnp.inf)
    row_max = None
    for it in range(_TOP_K):
        m = jnp.max(cur, axis=-1, keepdims=True)
        if it == 0:
            row_max = m
        cand = jnp.where(cur == m, lane, big)
        first = jnp.min(cand, axis=-1, keepdims=True)
        hit = lane == first
        mask = jnp.logical_or(mask, hit)
        cur = jnp.where(hit, neg_inf, cur)
    masked = jnp.where(mask, logits, 0.0)
    mx = jnp.maximum(row_max, 0.0)
    e = jnp.exp(masked - mx)
    o_ref[...] = e / jnp.sum(e, axis=-1, keepdims=True)


@jax.jit
def kernel(x, W, b):
    n_tok, d_model = x.shape
    n_exp = W.shape[1]
    b2 = b.reshape(1, n_exp)
    grid = (n_tok // _ROW_TILE,)
    return pl.pallas_call(
        _gating_body,
        grid=grid,
        in_specs=[
            pl.BlockSpec((_ROW_TILE, d_model), lambda i: (i, 0)),
            pl.BlockSpec((d_model, n_exp), lambda i: (0, 0)),
            pl.BlockSpec((1, n_exp), lambda i: (0, 0)),
        ],
        out_specs=pl.BlockSpec((_ROW_TILE, n_exp), lambda i: (i, 0)),
        out_shape=jax.ShapeDtypeStruct((n_tok, n_exp), jnp.float32),
        compiler_params=pltpu.CompilerParams(
            dimension_semantics=("arbitrary",),
        ),
    )(x, W, b2)
